# baseline (speedup 1.0000x reference)
import jax
import jax.numpy as jnp
from jax.experimental import pallas as pl
from jax.experimental.pallas import tpu as pltpu


def _decoder_kernel(seq_len, B, N, HID, OUT_DIM,
                    s_ref, h0_ref, w1g_ref, b1g_ref, w1c_ref, b1c_ref,
                    w2g_ref, b2g_ref, w2c_ref, b2c_ref, wp_ref, bp_ref,
                    out_ref, h_scr, cur_scr):
    t = pl.program_id(0)

    @pl.when(t == 0)
    def _init():
        h_scr[...] = h0_ref[...]
        cur_scr[...] = jnp.zeros((N, B), jnp.float32)

    S = s_ref[...]

    def matmul(a, b):
        return jax.lax.dot(a.astype(jnp.bfloat16), b.astype(jnp.bfloat16),
                           preferred_element_type=jnp.float32)

    def cell(inp_b, h_b, wg_ref, bg_ref, wc_ref, bc_ref):
        x0 = jnp.concatenate([inp_b, h_b], axis=1)
        x1 = matmul(S, x0)
        x2 = 2.0 * matmul(S, x1) - x0
        g = (matmul(x0, wg_ref[0]) + matmul(x1, wg_ref[1])
             + matmul(x2, wg_ref[2]) + bg_ref[...])
        g = jax.nn.sigmoid(g)
        r = g[:, :HID]
        u = g[:, HID:]
        x0c = jnp.concatenate([inp_b, r * h_b], axis=1)
        x1c = matmul(S, x0c)
        x2c = 2.0 * matmul(S, x1c) - x0c
        c = (matmul(x0c, wc_ref[0]) + matmul(x1c, wc_ref[1])
             + matmul(x2c, wc_ref[2]) + bc_ref[...])
        c = jnp.tanh(c)
        return u * h_b + (1.0 - u) * c

    cur = cur_scr[...]
    cols = []
    for b in range(B):
        inp1 = cur[:, b:b + 1]
        h0_b = h_scr[0, b]
        h0_new = cell(inp1, h0_b, w1g_ref, b1g_ref, w1c_ref, b1c_ref)
        h_scr[0, b] = h0_new
        h1_b = h_scr[1, b]
        h1_new = cell(h0_new, h1_b, w2g_ref, b2g_ref, w2c_ref, b2c_ref)
        h_scr[1, b] = h1_new
        cols.append(matmul(h1_new, wp_ref[...]) + bp_ref[...])
    proj = jnp.concatenate(cols, axis=1)
    cur_scr[...] = proj
    out_ref[0] = proj


def kernel(inputs, initial_hidden_state, supports, W1_gate, b1_gate,
           W1_cand, b1_cand, W2_gate, b2_gate, W2_cand, b2_cand, Wp, bp):
    seq_len, B = inputs.shape[0], inputs.shape[1]
    N = supports.shape[1]
    HID = Wp.shape[0]
    OUT_DIM = Wp.shape[1]
    num_layers = initial_hidden_state.shape[0]
    nm = 3

    S = supports[0]
    h0 = initial_hidden_state.reshape(num_layers, B, N, HID)
    in1 = OUT_DIM + HID
    in2 = HID + HID
    w1g = W1_gate.reshape(in1, nm, 2 * HID).transpose(1, 0, 2)
    w1c = W1_cand.reshape(in1, nm, HID).transpose(1, 0, 2)
    w2g = W2_gate.reshape(in2, nm, 2 * HID).transpose(1, 0, 2)
    w2c = W2_cand.reshape(in2, nm, HID).transpose(1, 0, 2)
    b1g = b1_gate.reshape(1, 2 * HID)
    b1c = b1_cand.reshape(1, HID)
    b2g = b2_gate.reshape(1, 2 * HID)
    b2c = b2_cand.reshape(1, HID)
    bp2 = bp.reshape(1, OUT_DIM)

    import functools
    body = functools.partial(_decoder_kernel, seq_len, B, N, HID, OUT_DIM)

    full = lambda shape: pl.BlockSpec(shape, lambda t: (0,) * len(shape))
    out = pl.pallas_call(
        body,
        grid=(seq_len,),
        in_specs=[
            full((N, N)),
            full((num_layers, B, N, HID)),
            full(w1g.shape), full(b1g.shape),
            full(w1c.shape), full(b1c.shape),
            full(w2g.shape), full(b2g.shape),
            full(w2c.shape), full(b2c.shape),
            full(Wp.shape), full(bp2.shape),
        ],
        out_specs=pl.BlockSpec((1, N, B), lambda t: (t, 0, 0)),
        out_shape=jax.ShapeDtypeStruct((seq_len, N, B), jnp.float32),
        scratch_shapes=[
            pltpu.VMEM((num_layers, B, N, HID), jnp.float32),
            pltpu.VMEM((N, B), jnp.float32),
        ],
        compiler_params=pltpu.CompilerParams(
            dimension_semantics=("arbitrary",),
        ),
    )(S, h0, w1g, b1g, w1c, b1c, w2g, b2g, w2c, b2c, Wp, bp2)

    return out.transpose(0, 2, 1).reshape(seq_len, B, N * OUT_DIM)


# speedup vs baseline: 2.1934x; 2.1934x over previous
import functools

import jax
import jax.numpy as jnp
from jax.experimental import pallas as pl
from jax.experimental.pallas import tpu as pltpu


def _decoder_kernel(B, N, HID, st_ref, h0i_ref, w1gh_ref, w1gi_ref,
                    b1g_ref, w1ch_ref, w1ci_ref, b1c_ref, w2g_ref, b2g_ref,
                    w2ci_ref, w2ch_ref, b2c_ref, wpt_ref, bp_ref,
                    out_ref, h0_scr, h1_scr, cur_scr):
    t = pl.program_id(0)

    @pl.when(t == 0)
    def _init():
        h0_scr[...] = h0i_ref[0]
        h1_scr[...] = h0i_ref[1]
        cur_scr[...] = jnp.zeros((8, B * N), jnp.float32)

    def matmul(a, b):
        return jax.lax.dot(a, b, preferred_element_type=jnp.float32)

    def diffuse(x):
        return jnp.concatenate(
            [matmul(x[:, b * N:(b + 1) * N], st_ref[...]) for b in range(B)],
            axis=1)

    h0 = h0_scr[...]
    h1 = h1_scr[...]

    y0 = jnp.concatenate([h0, cur_scr[...]], axis=0)
    y1 = diffuse(y0)
    y2 = 2.0 * diffuse(y1) - y0
    g = b1g_ref[...]
    c = b1c_ref[...]
    for k, yk in enumerate((y0, y1, y2)):
        g = g + matmul(w1gh_ref[k], yk[:HID]) + matmul(w1gi_ref[k], yk[HID:])
    g = jax.nn.sigmoid(g)
    r, u = g[:HID], g[HID:]
    rh0 = r * h0
    rh1 = diffuse(rh0)
    rh2 = 2.0 * diffuse(rh1) - rh0
    for k, (rhk, yk) in enumerate(((rh0, y0), (rh1, y1), (rh2, y2))):
        c = c + matmul(w1ch_ref[k], rhk) + matmul(w1ci_ref[k], yk[HID:])
    c = jnp.tanh(c)
    h0n = u * h0 + (1.0 - u) * c
    h0_scr[...] = h0n

    x0 = jnp.concatenate([h0n, h1], axis=0)
    x1 = diffuse(x0)
    x2 = 2.0 * diffuse(x1) - x0
    g = b2g_ref[...]
    c = b2c_ref[...]
    for k, xk in enumerate((x0, x1, x2)):
        g = g + matmul(w2g_ref[k], xk)
    g = jax.nn.sigmoid(g)
    r, u = g[:HID], g[HID:]
    rh0 = r * h1
    rh1 = diffuse(rh0)
    rh2 = 2.0 * diffuse(rh1) - rh0
    for k, (rhk, xk) in enumerate(((rh0, x0), (rh1, x1), (rh2, x2))):
        c = c + matmul(w2ci_ref[k], xk[:HID]) + matmul(w2ch_ref[k], rhk)
    c = jnp.tanh(c)
    h1n = u * h1 + (1.0 - u) * c
    h1_scr[...] = h1n

    proj = matmul(wpt_ref[...], h1n) + bp_ref[...]
    cur_scr[0:1] = proj
    out_ref[0] = proj


def kernel(inputs, initial_hidden_state, supports, W1_gate, b1_gate,
           W1_cand, b1_cand, W2_gate, b2_gate, W2_cand, b2_cand, Wp, bp):
    seq_len, B = inputs.shape[0], inputs.shape[1]
    N = supports.shape[1]
    HID = Wp.shape[0]
    OUT_DIM = Wp.shape[1]
    num_layers = initial_hidden_state.shape[0]
    nm = 3

    St = supports[0].T
    h0i = (initial_hidden_state.reshape(num_layers, B, N, HID)
           .transpose(0, 3, 1, 2).reshape(num_layers, HID, B * N))

    w1g = W1_gate.reshape(1 + HID, nm, 2 * HID)
    w1c = W1_cand.reshape(1 + HID, nm, HID)
    pad = jnp.zeros((7, nm, 2 * HID), jnp.float32)
    padc = jnp.zeros((7, nm, HID), jnp.float32)
    w1gh = w1g[1:].transpose(1, 2, 0)
    w1gi = jnp.concatenate([w1g[:1], pad], 0).transpose(1, 2, 0)
    w1ch = w1c[1:].transpose(1, 2, 0)
    w1ci = jnp.concatenate([w1c[:1], padc], 0).transpose(1, 2, 0)
    w2g = W2_gate.reshape(2 * HID, nm, 2 * HID).transpose(1, 2, 0)
    w2c = W2_cand.reshape(2 * HID, nm, HID)
    w2ci = w2c[:HID].transpose(1, 2, 0)
    w2ch = w2c[HID:].transpose(1, 2, 0)

    b1g = b1_gate.reshape(2 * HID, 1)
    b1c = b1_cand.reshape(HID, 1)
    b2g = b2_gate.reshape(2 * HID, 1)
    b2c = b2_cand.reshape(HID, 1)
    wpt = Wp.T
    bp2 = bp.reshape(1, 1)

    body = functools.partial(_decoder_kernel, B, N, HID)
    full = lambda shape: pl.BlockSpec(shape, lambda t: (0,) * len(shape))
    out = pl.pallas_call(
        body,
        grid=(seq_len,),
        in_specs=[
            full(St.shape), full(h0i.shape),
            full(w1gh.shape), full(w1gi.shape), full(b1g.shape),
            full(w1ch.shape), full(w1ci.shape), full(b1c.shape),
            full(w2g.shape), full(b2g.shape),
            full(w2ci.shape), full(w2ch.shape), full(b2c.shape),
            full(wpt.shape), full(bp2.shape),
        ],
        out_specs=pl.BlockSpec((1, 1, B * N), lambda t: (t, 0, 0)),
        out_shape=jax.ShapeDtypeStruct((seq_len, 1, B * N), jnp.float32),
        scratch_shapes=[
            pltpu.VMEM((HID, B * N), jnp.float32),
            pltpu.VMEM((HID, B * N), jnp.float32),
            pltpu.VMEM((8, B * N), jnp.float32),
        ],
        compiler_params=pltpu.CompilerParams(
            dimension_semantics=("arbitrary",),
        ),
    )(St, h0i, w1gh, w1gi, b1g, w1ch, w1ci, b1c, w2g, b2g, w2ci, w2ch, b2c,
      wpt, bp2)

    return out.reshape(seq_len, B, N * OUT_DIM)


# speedup vs baseline: 2.2012x; 1.0035x over previous
import functools

import jax
import jax.numpy as jnp
from jax.experimental import pallas as pl
from jax.experimental.pallas import tpu as pltpu


def _decoder_kernel(B, N, HID, st_ref, h0i_ref, w1gh_ref, w1gi_ref,
                    b1g_ref, w1ch_ref, w1ci_ref, b1c_ref, w2g_ref, b2g_ref,
                    w2ci_ref, w2ch_ref, b2c_ref, wpt_ref, bp_ref,
                    out_ref, h0_scr, h1_scr, cur_scr):
    t = pl.program_id(0)

    @pl.when(t == 0)
    def _init():
        h0_scr[...] = h0i_ref[0]
        h1_scr[...] = h0i_ref[1]
        cur_scr[...] = jnp.zeros((8, B * N), jnp.float32)

    def matmul(a, b):
        return jax.lax.dot(a.astype(jnp.bfloat16), b.astype(jnp.bfloat16),
                           preferred_element_type=jnp.float32)

    def diffuse(x):
        return jnp.concatenate(
            [matmul(x[:, b * N:(b + 1) * N], st_ref[...]) for b in range(B)],
            axis=1)

    h0 = h0_scr[...]
    h1 = h1_scr[...]

    y0 = jnp.concatenate([h0, cur_scr[...]], axis=0)
    y1 = diffuse(y0)
    y2 = 2.0 * diffuse(y1) - y0
    g = b1g_ref[...]
    c = b1c_ref[...]
    for k, yk in enumerate((y0, y1, y2)):
        g = g + matmul(w1gh_ref[k], yk[:HID]) + matmul(w1gi_ref[k], yk[HID:])
    g = jax.nn.sigmoid(g)
    r, u = g[:HID], g[HID:]
    rh0 = r * h0
    rh1 = diffuse(rh0)
    rh2 = 2.0 * diffuse(rh1) - rh0
    for k, (rhk, yk) in enumerate(((rh0, y0), (rh1, y1), (rh2, y2))):
        c = c + matmul(w1ch_ref[k], rhk) + matmul(w1ci_ref[k], yk[HID:])
    c = jnp.tanh(c)
    h0n = u * h0 + (1.0 - u) * c
    h0_scr[...] = h0n

    x0 = jnp.concatenate([h0n, h1], axis=0)
    x1 = diffuse(x0)
    x2 = 2.0 * diffuse(x1) - x0
    g = b2g_ref[...]
    c = b2c_ref[...]
    for k, xk in enumerate((x0, x1, x2)):
        g = g + matmul(w2g_ref[k], xk)
    g = jax.nn.sigmoid(g)
    r, u = g[:HID], g[HID:]
    rh0 = r * h1
    rh1 = diffuse(rh0)
    rh2 = 2.0 * diffuse(rh1) - rh0
    for k, (rhk, xk) in enumerate(((rh0, x0), (rh1, x1), (rh2, x2))):
        c = c + matmul(w2ci_ref[k], xk[:HID]) + matmul(w2ch_ref[k], rhk)
    c = jnp.tanh(c)
    h1n = u * h1 + (1.0 - u) * c
    h1_scr[...] = h1n

    proj = matmul(wpt_ref[...], h1n) + bp_ref[...]
    cur_scr[0:1] = proj
    out_ref[0] = proj


def kernel(inputs, initial_hidden_state, supports, W1_gate, b1_gate,
           W1_cand, b1_cand, W2_gate, b2_gate, W2_cand, b2_cand, Wp, bp):
    seq_len, B = inputs.shape[0], inputs.shape[1]
    N = supports.shape[1]
    HID = Wp.shape[0]
    OUT_DIM = Wp.shape[1]
    num_layers = initial_hidden_state.shape[0]
    nm = 3

    St = supports[0].T
    h0i = (initial_hidden_state.reshape(num_layers, B, N, HID)
           .transpose(0, 3, 1, 2).reshape(num_layers, HID, B * N))

    w1g = W1_gate.reshape(1 + HID, nm, 2 * HID)
    w1c = W1_cand.reshape(1 + HID, nm, HID)
    pad = jnp.zeros((7, nm, 2 * HID), jnp.float32)
    padc = jnp.zeros((7, nm, HID), jnp.float32)
    w1gh = w1g[1:].transpose(1, 2, 0)
    w1gi = jnp.concatenate([w1g[:1], pad], 0).transpose(1, 2, 0)
    w1ch = w1c[1:].transpose(1, 2, 0)
    w1ci = jnp.concatenate([w1c[:1], padc], 0).transpose(1, 2, 0)
    w2g = W2_gate.reshape(2 * HID, nm, 2 * HID).transpose(1, 2, 0)
    w2c = W2_cand.reshape(2 * HID, nm, HID)
    w2ci = w2c[:HID].transpose(1, 2, 0)
    w2ch = w2c[HID:].transpose(1, 2, 0)

    b1g = b1_gate.reshape(2 * HID, 1)
    b1c = b1_cand.reshape(HID, 1)
    b2g = b2_gate.reshape(2 * HID, 1)
    b2c = b2_cand.reshape(HID, 1)
    wpt = Wp.T
    bp2 = bp.reshape(1, 1)

    body = functools.partial(_decoder_kernel, B, N, HID)
    full = lambda shape: pl.BlockSpec(shape, lambda t: (0,) * len(shape))
    out = pl.pallas_call(
        body,
        grid=(seq_len,),
        in_specs=[
            full(St.shape), full(h0i.shape),
            full(w1gh.shape), full(w1gi.shape), full(b1g.shape),
            full(w1ch.shape), full(w1ci.shape), full(b1c.shape),
            full(w2g.shape), full(b2g.shape),
            full(w2ci.shape), full(w2ch.shape), full(b2c.shape),
            full(wpt.shape), full(bp2.shape),
        ],
        out_specs=pl.BlockSpec((1, 1, B * N), lambda t: (t, 0, 0)),
        out_shape=jax.ShapeDtypeStruct((seq_len, 1, B * N), jnp.float32),
        scratch_shapes=[
            pltpu.VMEM((HID, B * N), jnp.float32),
            pltpu.VMEM((HID, B * N), jnp.float32),
            pltpu.VMEM((8, B * N), jnp.float32),
        ],
        compiler_params=pltpu.CompilerParams(
            dimension_semantics=("arbitrary",),
        ),
    )(St, h0i, w1gh, w1gi, b1g, w1ch, w1ci, b1c, w2g, b2g, w2ci, w2ch, b2c,
      wpt, bp2)

    return out.reshape(seq_len, B, N * OUT_DIM)
